# baseline (speedup 1.0000x reference)
import functools

import jax
import jax.numpy as jnp
from jax import lax
from jax.experimental import pallas as pl
from jax.experimental.pallas import tpu as pltpu
from jax.experimental.pallas import tpu_sc as plsc

BATCH = 16384
DIM = 64
LANES = 16


def _emb_mul_body(table_hbm, lat_hbm, idx_hbm, out_hbm,
                  idx_v, rows_v, lat_v, gsem, lsem, nc):
    wid = lax.axis_index("s") * nc + lax.axis_index("c")
    b_per_w = BATCH // (nc * 16)
    base = wid * b_per_w

    pltpu.sync_copy(idx_hbm.at[pl.ds(base, b_per_w)], idx_v)
    gather = pltpu.async_copy(table_hbm.at[idx_v], rows_v, gsem)
    latcp = pltpu.async_copy(lat_hbm.at[pl.ds(base, b_per_w)], lat_v, lsem)
    gather.wait()
    latcp.wait()

    def row(i, _):
        for j in range(DIM // LANES):
            sl = pl.ds(j * LANES, LANES)
            rows_v[i, sl] = rows_v[i, sl] * lat_v[i, sl]
        return 0

    lax.fori_loop(0, b_per_w, row, 0)
    pltpu.sync_copy(rows_v, out_hbm.at[pl.ds(base, b_per_w)])


def kernel(latent, label, emb_table):
    info = plsc.get_sparse_core_info()
    nc = info.num_cores
    b_per_w = BATCH // (nc * info.num_subcores)
    mesh = plsc.VectorSubcoreMesh(core_axis_name="c", subcore_axis_name="s")
    fn = pl.kernel(
        functools.partial(_emb_mul_body, nc=nc),
        mesh=mesh,
        out_type=jax.ShapeDtypeStruct((BATCH, DIM), jnp.float32),
        scratch_types=[
            pltpu.VMEM((b_per_w,), jnp.int32),
            pltpu.VMEM((b_per_w, DIM), jnp.float32),
            pltpu.VMEM((b_per_w, DIM), jnp.float32),
            pltpu.SemaphoreType.DMA,
            pltpu.SemaphoreType.DMA,
        ],
        compiler_params=pltpu.CompilerParams(use_tc_tiling_on_sc=False),
    )
    return fn(emb_table, latent, label.astype(jnp.int32))


# speedup vs baseline: 1.6867x; 1.6867x over previous
import functools

import jax
import jax.numpy as jnp
from jax import lax
from jax.experimental import pallas as pl
from jax.experimental.pallas import tpu as pltpu
from jax.experimental.pallas import tpu_sc as plsc

BATCH = 16384
DIM = 64
LANES = 16
CHUNK = 256


def _emb_mul_body(table_hbm, lat_hbm, idx_hbm, out_hbm,
                  idx_sh, idx_s, rows_v, lat_v, gsem, lsem, nc):
    wid = lax.axis_index("s") * nc + lax.axis_index("c")
    b_per_w = BATCH // (nc * 16)
    base = wid * b_per_w

    pltpu.sync_copy(idx_hbm.at[pl.ds(base, b_per_w)], idx_sh.at[wid])
    pltpu.sync_copy(idx_sh.at[wid], idx_s)

    for r in range(b_per_w // CHUNK):
        cbase = r * CHUNK
        latcp = pltpu.async_copy(
            lat_hbm.at[pl.ds(base + cbase, CHUNK)], lat_v, lsem)

        def fire(i, _):
            row = idx_s[cbase + i]
            pltpu.async_copy(table_hbm.at[pl.ds(row, 1)],
                             rows_v.at[pl.ds(i, 1)], gsem)
            return 0

        lax.fori_loop(0, CHUNK, fire, 0)
        pltpu.make_async_copy(
            table_hbm.at[pl.ds(0, CHUNK)], rows_v, gsem).wait()
        latcp.wait()

        def row(i, _):
            for j in range(DIM // LANES):
                sl = pl.ds(j * LANES, LANES)
                rows_v[i, sl] = rows_v[i, sl] * lat_v[i, sl]
            return 0

        lax.fori_loop(0, CHUNK, row, 0)
        pltpu.sync_copy(rows_v, out_hbm.at[pl.ds(base + cbase, CHUNK)])


def kernel(latent, label, emb_table):
    info = plsc.get_sparse_core_info()
    nc = info.num_cores
    b_per_w = BATCH // (nc * info.num_subcores)
    mesh = plsc.VectorSubcoreMesh(core_axis_name="c", subcore_axis_name="s")
    fn = pl.kernel(
        functools.partial(_emb_mul_body, nc=nc),
        mesh=mesh,
        out_type=jax.ShapeDtypeStruct((BATCH, DIM), jnp.float32),
        scratch_types=[
            pltpu.VMEM_SHARED((32, b_per_w), jnp.int32),
            pltpu.SMEM((b_per_w,), jnp.int32),
            pltpu.VMEM((CHUNK, DIM), jnp.float32),
            pltpu.VMEM((CHUNK, DIM), jnp.float32),
            pltpu.SemaphoreType.DMA,
            pltpu.SemaphoreType.DMA,
        ],
    )
    return fn(emb_table, latent, label.astype(jnp.int32))
